# baseline (speedup 1.0000x reference)
import functools

import jax
import jax.numpy as jnp
from jax import lax
from jax.experimental import pallas as pl
from jax.experimental.pallas import tpu as pltpu
from jax.experimental.pallas import tpu_sc as plsc

NN = 10000
EE = 160000
CC = 64
RBF = 8
NGR = 16
RCUT = 5.0
AVGN = 16.0

NCORE = 2
NSUB = 16

CH = 64
NCHUNK = EE // CH

BE = 1280
BN = 1000

_F32 = jnp.float32


def _dot(a, b):
    return lax.dot_general(a, b, (((1,), (0,)), ((), ())),
                           preferred_element_type=_F32,
                           precision=lax.Precision.DEFAULT)


def _silu(x):
    return x * (1.0 / (1.0 + jnp.exp(-x)))


VCH = 1280
NVCH = EE // VCH


def _vec_body(src_hbm, dst_hbm, pos_hbm, vt_hbm, pos_v, sidx, didx, vbuf):
    c = lax.axis_index("c")
    s = lax.axis_index("s")
    wid = s * NCORE + c

    pltpu.sync_copy(pos_hbm, pos_v)
    zero16 = jnp.zeros((16,), _F32)

    def zrow(g, _=None):
        for j in range(3, 8):
            vbuf[j, pl.ds(g * 16, 16)] = zero16

    plsc.parallel_loop(0, VCH // 16, unroll=4)(zrow)

    def chunk(i):
        k = i * (NCORE * NSUB) + wid

        @pl.when(k < NVCH)
        def _():
            base = k * VCH
            pltpu.sync_copy(src_hbm.at[pl.ds(base, VCH)], sidx)
            pltpu.sync_copy(dst_hbm.at[pl.ds(base, VCH)], didx)

            def grp(g, _=None):
                sv = sidx[pl.ds(g * 16, 16)] * 8
                dv = didx[pl.ds(g * 16, 16)] * 8
                for j in range(3):
                    ps = plsc.load_gather(pos_v, [sv + j])
                    pd = plsc.load_gather(pos_v, [dv + j])
                    vbuf[j, pl.ds(g * 16, 16)] = pd - ps

            plsc.parallel_loop(0, VCH // 16, unroll=4)(grp)
            pltpu.sync_copy(vbuf, vt_hbm.at[:, pl.ds(base, VCH)])

    pl.loop(0, pl.cdiv(NVCH, NCORE * NSUB))(chunk)


def _sc_vec(src, dst, pos1d):
    mesh = plsc.VectorSubcoreMesh(core_axis_name="c", subcore_axis_name="s")
    f = functools.partial(
        pl.kernel, _vec_body, mesh=mesh,
        out_type=jax.ShapeDtypeStruct((8, EE), _F32),
        scratch_types=[pltpu.VMEM((NN * 8,), _F32),
                       pltpu.VMEM((VCH,), jnp.int32),
                       pltpu.VMEM((VCH,), jnp.int32),
                       pltpu.VMEM((8, VCH), _F32)],
        compiler_params=pltpu.CompilerParams(needs_layout_passes=False),
    )
    return f()(src, dst, pos1d)


def _layer_body(src_hbm, dst_hbm, hmv_hbm, r_hbm, u_hbm, zeros_hbm, a_hbm,
                idx6, rbuf2, ubuf2, hsb2,
                acc,
                semi_a, semi_b, seml_a, seml_b,
                semg_a, semg_b, sems_a, sems_b):
    c = lax.axis_index("c")
    s = lax.axis_index("s")
    sidx_a, sidx_b = idx6.at[0], idx6.at[1]
    didx0, didx1, didx2, didx3 = (idx6.at[2], idx6.at[3],
                                  idx6.at[4], idx6.at[5])
    rbuf_a, rbuf_b = rbuf2.at[0], rbuf2.at[1]
    ubuf_a, ubuf_b = ubuf2.at[0], ubuf2.at[1]
    hsb_a, hsb_b = hsb2.at[0], hsb2.at[1]
    mbuf_a, mbuf_b = hsb_a, hsb_b
    nrows = 624
    ntail = NN - NSUB * nrows

    zbase = s * nrows
    pltpu.sync_copy(zeros_hbm.at[pl.ds(zbase, nrows)],
                    acc.at[pl.ds(zbase, nrows)])

    @pl.when(s == 0)
    def _():
        pltpu.sync_copy(zeros_hbm.at[pl.ds(NSUB * nrows, ntail)],
                        acc.at[pl.ds(NSUB * nrows, ntail)])

    plsc.subcore_barrier()

    slot_a = (sidx_a, rbuf_a, ubuf_a, hsb_a, mbuf_a,
              semi_a, seml_a, semg_a, sems_a)
    slot_b = (sidx_b, rbuf_b, ubuf_b, hsb_b, mbuf_b,
              semi_b, seml_b, semg_b, sems_b)

    def loads(i, sl, dx):
        sidx, rbuf, ubuf, _, _, semi, seml, _, _ = sl
        k = i * NSUB + s
        base = k * CH
        pltpu.async_copy(src_hbm.at[pl.ds(base, CH)], sidx, semi)
        pltpu.async_copy(dst_hbm.at[pl.ds(base, CH)], dx, seml)
        pltpu.async_copy(r_hbm.at[pl.ds(base, CH)], rbuf, seml)
        pltpu.async_copy(u_hbm.at[pl.ds(base * 16, CH * 16)], ubuf, seml)

    def gather_start(sl):
        sidx, _, _, hsb, _, semi, _, semg, _ = sl
        pltpu.make_async_copy(src_hbm.at[pl.ds(0, CH)], sidx, semi).wait()
        pltpu.async_copy(hmv_hbm.at[sidx], hsb, semg)

    def scatter_drain(sl):
        mbuf, sems = sl[4], sl[8]
        pltpu.make_async_copy(zeros_hbm.at[pl.ds(0, CH)], mbuf, sems).wait()

    def compute(sl):
        _, rbuf, ubuf, hsb, mbuf, _, _, _, _ = sl

        @pl.when(c == 0)
        def _():
            def edge(e, _=None):
                uvec = ubuf[pl.ds(e * 16, 16)]
                ux = uvec[0]
                for q in range(CC // 16):
                    rv = rbuf[e, pl.ds(q * 16, 16)]
                    hm = hsb[e, pl.ds(q * 16, 16)]
                    hv = hsb[e, pl.ds(CC + q * 16, 16)]
                    mbuf[e, pl.ds(q * 16, 16)] = rv * hm
                    mbuf[e, pl.ds(CC + q * 16, 16)] = rv * hv * ux

            plsc.parallel_loop(0, CH, unroll=4)(edge)

        @pl.when(c == 1)
        def _():
            def edge(e, _=None):
                uvec = ubuf[pl.ds(e * 16, 16)]
                uy = uvec[1]
                uz = uvec[2]
                for q in range(CC // 16):
                    rv = rbuf[e, pl.ds(q * 16, 16)]
                    hv = hsb[e, pl.ds(CC + q * 16, 16)]
                    t = rv * hv
                    mbuf[e, pl.ds(q * 16, 16)] = t * uy
                    mbuf[e, pl.ds(CC + q * 16, 16)] = t * uz

            plsc.parallel_loop(0, CH, unroll=4)(edge)

    def process(i, sl, other, dx, dx2, swait_cond):
        _, rbuf, ubuf, hsb, mbuf, _, seml, semg, sems = sl
        k = i * NSUB + s

        @pl.when(k < NCHUNK)
        def _():
            pltpu.make_async_copy(dst_hbm.at[pl.ds(0, CH)], dx, seml).wait()
            pltpu.make_async_copy(r_hbm.at[pl.ds(0, CH)], rbuf, seml).wait()
            pltpu.make_async_copy(u_hbm.at[pl.ds(0, CH * 16)], ubuf,
                                  seml).wait()
            pltpu.make_async_copy(
                hmv_hbm.at[pl.ds(0, CH)], hsb, semg).wait()
            compute(sl)

            @pl.when(k + NSUB < NCHUNK)
            def _():
                if swait_cond is None:
                    scatter_drain(other)
                else:
                    @pl.when(swait_cond)
                    def _():
                        scatter_drain(other)
                gather_start(other)

            pltpu.async_copy(mbuf, acc.at[dx], sems, add=True)

            @pl.when(k + 2 * NSUB < NCHUNK)
            def _():
                loads(i + 2, sl, dx2)

    def chunk_sync(i):
        k = i * NSUB + s

        @pl.when(k < NCHUNK)
        def _():
            base = k * CH
            pltpu.sync_copy(src_hbm.at[pl.ds(base, CH)], sidx_a)
            pltpu.sync_copy(dst_hbm.at[pl.ds(base, CH)], didx0)
            pltpu.sync_copy(r_hbm.at[pl.ds(base, CH)], rbuf_a)
            pltpu.sync_copy(u_hbm.at[pl.ds(base * 16, CH * 16)], ubuf_a)
            pltpu.async_copy(hmv_hbm.at[sidx_a], hsb_a, semg_a).wait()
            compute(slot_a)
            pltpu.sync_copy(mbuf_a, acc.at[didx0], add=True)

    pl.loop(0, pl.cdiv(NCHUNK, NSUB))(chunk_sync)

    plsc.subcore_barrier()
    pltpu.sync_copy(acc.at[pl.ds(zbase, nrows)],
                    a_hbm.at[c, pl.ds(zbase, nrows)])

    @pl.when(s == 0)
    def _():
        pltpu.sync_copy(acc.at[pl.ds(NSUB * nrows, ntail)],
                        a_hbm.at[c, pl.ds(NSUB * nrows, ntail)])


def _sc_layer(src, dst, hmv, r_arr, u16, zeros_acc):
    mesh = plsc.VectorSubcoreMesh(core_axis_name="c", subcore_axis_name="s")
    f = functools.partial(
        pl.kernel, _layer_body, mesh=mesh,
        out_type=jax.ShapeDtypeStruct((2, NN, 2 * CC), _F32),
        scratch_types=[pltpu.VMEM((6, CH), jnp.int32),
                       pltpu.VMEM((2, CH, CC), _F32),
                       pltpu.VMEM((2, CH * 16), _F32),
                       pltpu.VMEM((2, CH, 2 * CC), _F32),
                       pltpu.VMEM_SHARED((NN, 2 * CC), _F32),
                       pltpu.SemaphoreType.DMA,
                       pltpu.SemaphoreType.DMA,
                       pltpu.SemaphoreType.DMA,
                       pltpu.SemaphoreType.DMA,
                       pltpu.SemaphoreType.DMA,
                       pltpu.SemaphoreType.DMA,
                       pltpu.SemaphoreType.DMA,
                       pltpu.SemaphoreType.DMA],
    )
    return f()(src, dst, hmv, r_arr, u16.reshape(EE * 16), zeros_acc)


def _node0_body(spec_ref, we_ref, wm_ref, wv_ref, out_ref):
    oh = (spec_ref[...] == lax.broadcasted_iota(jnp.int32, (1, 16), 1))
    h0 = _dot(oh.astype(_F32), we_ref[...])
    out_ref[...] = jnp.concatenate(
        [_dot(h0, wm_ref[...]), _dot(h0, wv_ref[...])], axis=1)


def _tc_node0(spec, we16, wm, wv):
    return pl.pallas_call(
        _node0_body,
        grid=(NN // BN,),
        in_specs=[pl.BlockSpec((BN, 1), lambda i: (i, 0)),
                  pl.BlockSpec((16, CC), lambda i: (0, 0)),
                  pl.BlockSpec((CC, CC), lambda i: (0, 0)),
                  pl.BlockSpec((CC, CC), lambda i: (0, 0))],
        out_specs=pl.BlockSpec((BN, 2 * CC), lambda i: (i, 0)),
        out_shape=jax.ShapeDtypeStruct((NN, 2 * CC), _F32),
    )(spec, we16, wm, wv)


def _geom_body(vt_ref, wr1_ref, wr2_ref, r_ref, u_ref):
    v = vt_ref[...]
    r2 = jnp.sum(v * v, axis=0, keepdims=True) + 1e-12
    r = jnp.sqrt(r2)
    rinv = 1.0 / r
    uT = v * rinv

    def t8(a):
        return jnp.transpose(a, (1, 0))

    u_ref[...] = jnp.concatenate(
        [t8(uT), jnp.zeros((uT.shape[1], 8), _F32)], axis=1)
    x = r / RCUT
    nv = lax.broadcasted_iota(jnp.int32, (RBF, 1), 0).astype(_F32) + 1.0
    bes = jnp.sqrt(2.0 / RCUT) * jnp.sin(nv * (jnp.pi * x)) * rinv
    x6 = x * x * x
    x6 = x6 * x6
    fcut = (1.0 - 28.0 * x6 + 48.0 * x6 * x - 21.0 * x6 * x * x)
    fcut = jnp.where(x < 1.0, fcut, 0.0)
    rb = t8(bes * fcut)
    r_ref[...] = _dot(_silu(_dot(rb, wr1_ref[...])), wr2_ref[...])


def _tc_geom(vecT, wr1, wr2):
    return pl.pallas_call(
        _geom_body,
        grid=(EE // BE,),
        in_specs=[pl.BlockSpec((8, BE), lambda i: (0, i)),
                  pl.BlockSpec((RBF, CC), lambda i: (0, 0)),
                  pl.BlockSpec((CC, CC), lambda i: (0, 0))],
        out_specs=[pl.BlockSpec((BE, CC), lambda i: (i, 0)),
                   pl.BlockSpec((BE, 16), lambda i: (i, 0))],
        out_shape=[jax.ShapeDtypeStruct((EE, CC), _F32),
                   jax.ShapeDtypeStruct((EE, 16), _F32)],
    )(vecT, wr1, wr2)


def _node_body(a_ref, wu_ref, wm_ref, wv_ref, out_ref):
    a = a_ref[...] * (1.0 / AVGN)
    a1x = a[0, :, CC:]
    a1y = a[1, :, :CC]
    a1z = a[1, :, CC:]
    inv = a[0, :, :CC] + a1x * a1x + a1y * a1y + a1z * a1z
    h = _silu(_dot(inv, wu_ref[...]))
    out_ref[...] = jnp.concatenate(
        [_dot(h, wm_ref[...]), _dot(h, wv_ref[...])], axis=1)


def _tc_node(a_arr, wu, wm, wv):
    return pl.pallas_call(
        _node_body,
        grid=(NN // BN,),
        in_specs=[pl.BlockSpec((2, BN, 2 * CC), lambda i: (0, i, 0)),
                  pl.BlockSpec((CC, CC), lambda i: (0, 0)),
                  pl.BlockSpec((CC, CC), lambda i: (0, 0)),
                  pl.BlockSpec((CC, CC), lambda i: (0, 0))],
        out_specs=pl.BlockSpec((BN, 2 * CC), lambda i: (i, 0)),
        out_shape=jax.ShapeDtypeStruct((NN, 2 * CC), _F32),
    )(a_arr, wu, wm, wv)


def _final_body(a_ref, wu_ref, w1_ref, w2_ref, spec_ref, ae_ref, bat_ref,
                out_ref):
    @pl.when(pl.program_id(0) == 0)
    def _():
        out_ref[...] = jnp.zeros_like(out_ref)

    a = a_ref[...] * (1.0 / AVGN)
    a1x = a[0, :, CC:]
    a1y = a[1, :, :CC]
    a1z = a[1, :, CC:]
    inv = a[0, :, :CC] + a1x * a1x + a1y * a1y + a1z * a1z
    h = _silu(_dot(inv, wu_ref[...]))
    e = _dot(_silu(_dot(h, w1_ref[...])), w2_ref[...])
    oh_s = (spec_ref[...] == lax.broadcasted_iota(jnp.int32, (1, 16), 1))
    e = e + _dot(oh_s.astype(_F32), ae_ref[...])
    oh_b = (bat_ref[...] == lax.broadcasted_iota(jnp.int32, (1, 16), 1))
    out_ref[...] += jnp.sum(oh_b.astype(_F32) * e, axis=0, keepdims=True)


def _tc_final(a_arr, wu, w1, w2, spec, ae16, bat):
    return pl.pallas_call(
        _final_body,
        grid=(NN // BN,),
        in_specs=[pl.BlockSpec((2, BN, 2 * CC), lambda i: (0, i, 0)),
                  pl.BlockSpec((CC, CC), lambda i: (0, 0)),
                  pl.BlockSpec((CC, 16), lambda i: (0, 0)),
                  pl.BlockSpec((16, 1), lambda i: (0, 0)),
                  pl.BlockSpec((BN, 1), lambda i: (i, 0)),
                  pl.BlockSpec((16, 1), lambda i: (0, 0)),
                  pl.BlockSpec((BN, 1), lambda i: (i, 0))],
        out_specs=pl.BlockSpec((1, 16), lambda i: (0, 0)),
        out_shape=jax.ShapeDtypeStruct((1, 16), _F32),
    )(a_arr, wu, w1, w2, spec, ae16, bat)


def kernel(positions, edge_index, species, batch,
           W_embed, W_r1, W_r2,
           W_msg0, W_vec0, W_up0,
           W_msg1, W_vec1, W_up1,
           W_ro1, W_ro2, atomic_E):
    src = edge_index[0].astype(jnp.int32)
    dst = edge_index[1].astype(jnp.int32)
    pos1d = jnp.pad(positions.astype(_F32), ((0, 0), (0, 5))).reshape(-1)
    we16 = jnp.pad(W_embed, ((0, 6), (0, 0)))
    ae16 = jnp.pad(atomic_E, (0, 6)).reshape(16, 1)
    spec = species.astype(jnp.int32).reshape(NN, 1)
    bat = batch.astype(jnp.int32).reshape(NN, 1)
    zeros_acc = jnp.zeros((NN, 2 * CC), _F32)

    hmv = _tc_node0(spec, we16, W_msg0, W_vec0)
    vecT = _sc_vec(src, dst, pos1d)
    r_arr, u8 = _tc_geom(vecT, W_r1, W_r2)

    a_arr = _sc_layer(src, dst, hmv, r_arr, u8, zeros_acc)
    hmv = _tc_node(a_arr, W_up0, W_msg1, W_vec1)
    a_arr = _sc_layer(src, dst, hmv, r_arr, u8, zeros_acc)

    energy = _tc_final(a_arr, W_up1, W_ro1, W_ro2, spec, ae16, bat)
    return energy.reshape(NGR)


# speedup vs baseline: 1.9336x; 1.9336x over previous
import functools

import jax
import jax.numpy as jnp
from jax import lax
from jax.experimental import pallas as pl
from jax.experimental.pallas import tpu as pltpu
from jax.experimental.pallas import tpu_sc as plsc

NN = 10000
EE = 160000
CC = 64
RBF = 8
NGR = 16
RCUT = 5.0
AVGN = 16.0

NCORE = 2
NSUB = 16

CH = 64
NCHUNK = EE // CH

BE = 1280
BN = 1000

_F32 = jnp.float32


def _dot(a, b):
    return lax.dot_general(a, b, (((1,), (0,)), ((), ())),
                           preferred_element_type=_F32,
                           precision=lax.Precision.DEFAULT)


def _silu(x):
    return x * (1.0 / (1.0 + jnp.exp(-x)))


VCH = 1280
NVCH = EE // VCH


def _vec_body(src_hbm, dst_hbm, pos_hbm, vt_hbm, pos_v, sidx, didx, vbuf):
    c = lax.axis_index("c")
    s = lax.axis_index("s")
    wid = s * NCORE + c

    pltpu.sync_copy(pos_hbm, pos_v)
    zero16 = jnp.zeros((16,), _F32)

    def zrow(g, _=None):
        for j in range(3, 8):
            vbuf[j, pl.ds(g * 16, 16)] = zero16

    plsc.parallel_loop(0, VCH // 16, unroll=4)(zrow)

    def chunk(i):
        k = i * (NCORE * NSUB) + wid

        @pl.when(k < NVCH)
        def _():
            base = k * VCH
            pltpu.sync_copy(src_hbm.at[pl.ds(base, VCH)], sidx)
            pltpu.sync_copy(dst_hbm.at[pl.ds(base, VCH)], didx)

            def grp(g, _=None):
                sv = sidx[pl.ds(g * 16, 16)] * 8
                dv = didx[pl.ds(g * 16, 16)] * 8
                for j in range(3):
                    ps = plsc.load_gather(pos_v, [sv + j])
                    pd = plsc.load_gather(pos_v, [dv + j])
                    vbuf[j, pl.ds(g * 16, 16)] = pd - ps

            plsc.parallel_loop(0, VCH // 16, unroll=4)(grp)
            pltpu.sync_copy(vbuf, vt_hbm.at[:, pl.ds(base, VCH)])

    pl.loop(0, pl.cdiv(NVCH, NCORE * NSUB))(chunk)


def _sc_vec(src, dst, pos1d):
    mesh = plsc.VectorSubcoreMesh(core_axis_name="c", subcore_axis_name="s")
    f = functools.partial(
        pl.kernel, _vec_body, mesh=mesh,
        out_type=jax.ShapeDtypeStruct((8, EE), _F32),
        scratch_types=[pltpu.VMEM((NN * 8,), _F32),
                       pltpu.VMEM((VCH,), jnp.int32),
                       pltpu.VMEM((VCH,), jnp.int32),
                       pltpu.VMEM((8, VCH), _F32)],
        compiler_params=pltpu.CompilerParams(needs_layout_passes=False),
    )
    return f()(src, dst, pos1d)


def _layer_body(src_hbm, dst_hbm, hmv_hbm, r_hbm, u_hbm, zeros_hbm, a_hbm,
                idx6, rbuf2, ubuf2, hsb2,
                acc,
                semi_a, semi_b, seml_a, seml_b,
                semg_a, semg_b, sems_a, sems_b):
    c = lax.axis_index("c")
    s = lax.axis_index("s")
    sidx_a, sidx_b = idx6.at[0], idx6.at[1]
    didx0, didx1, didx2, didx3 = (idx6.at[2], idx6.at[3],
                                  idx6.at[4], idx6.at[5])
    rbuf_a, rbuf_b = rbuf2.at[0], rbuf2.at[1]
    ubuf_a, ubuf_b = ubuf2.at[0], ubuf2.at[1]
    hsb_a, hsb_b = hsb2.at[0], hsb2.at[1]
    mbuf_a, mbuf_b = hsb_a, hsb_b
    nrows = 624
    ntail = NN - NSUB * nrows

    zbase = s * nrows
    pltpu.sync_copy(zeros_hbm.at[pl.ds(zbase, nrows)],
                    acc.at[pl.ds(zbase, nrows)])

    @pl.when(s == 0)
    def _():
        pltpu.sync_copy(zeros_hbm.at[pl.ds(NSUB * nrows, ntail)],
                        acc.at[pl.ds(NSUB * nrows, ntail)])

    plsc.subcore_barrier()

    slot_a = (sidx_a, rbuf_a, ubuf_a, hsb_a, mbuf_a,
              semi_a, seml_a, semg_a, sems_a)
    slot_b = (sidx_b, rbuf_b, ubuf_b, hsb_b, mbuf_b,
              semi_b, seml_b, semg_b, sems_b)

    def loads(i, sl, dx):
        sidx, rbuf, ubuf, _, _, semi, seml, _, _ = sl
        k = i * NSUB + s
        base = k * CH
        pltpu.async_copy(src_hbm.at[pl.ds(base, CH)], sidx, semi)
        pltpu.async_copy(dst_hbm.at[pl.ds(base, CH)], dx, seml)
        pltpu.async_copy(r_hbm.at[pl.ds(base, CH)], rbuf, seml)
        pltpu.async_copy(u_hbm.at[pl.ds(base * 16, CH * 16)], ubuf, seml)

    def gather_start(sl):
        sidx, _, _, hsb, _, semi, _, semg, _ = sl
        pltpu.make_async_copy(src_hbm.at[pl.ds(0, CH)], sidx, semi).wait()
        pltpu.async_copy(hmv_hbm.at[sidx], hsb, semg)

    def scatter_drain(sl):
        mbuf, sems = sl[4], sl[8]
        pltpu.make_async_copy(mbuf, acc.at[idx6.at[2]], sems).wait()

    def compute(sl):
        _, rbuf, ubuf, hsb, mbuf, _, _, _, _ = sl

        @pl.when(c == 0)
        def _():
            def edge(e, _=None):
                uvec = ubuf[pl.ds(e * 16, 16)]
                ux = uvec[0]
                for q in range(CC // 16):
                    rv = rbuf[e, pl.ds(q * 16, 16)]
                    hm = hsb[e, pl.ds(q * 16, 16)]
                    hv = hsb[e, pl.ds(CC + q * 16, 16)]
                    mbuf[e, pl.ds(q * 16, 16)] = rv * hm
                    mbuf[e, pl.ds(CC + q * 16, 16)] = rv * hv * ux

            plsc.parallel_loop(0, CH, unroll=4)(edge)

        @pl.when(c == 1)
        def _():
            def edge(e, _=None):
                uvec = ubuf[pl.ds(e * 16, 16)]
                uy = uvec[1]
                uz = uvec[2]
                for q in range(CC // 16):
                    rv = rbuf[e, pl.ds(q * 16, 16)]
                    hv = hsb[e, pl.ds(CC + q * 16, 16)]
                    t = rv * hv
                    mbuf[e, pl.ds(q * 16, 16)] = t * uy
                    mbuf[e, pl.ds(CC + q * 16, 16)] = t * uz

            plsc.parallel_loop(0, CH, unroll=4)(edge)

    def process(i, sl, other, dx, dx2, swait_cond):
        _, rbuf, ubuf, hsb, mbuf, _, seml, semg, sems = sl
        k = i * NSUB + s

        @pl.when(k < NCHUNK)
        def _():
            pltpu.make_async_copy(dst_hbm.at[pl.ds(0, CH)], dx, seml).wait()
            pltpu.make_async_copy(r_hbm.at[pl.ds(0, CH)], rbuf, seml).wait()
            pltpu.make_async_copy(u_hbm.at[pl.ds(0, CH * 16)], ubuf,
                                  seml).wait()
            pltpu.make_async_copy(hmv_hbm.at[sl[0]], hsb, semg).wait()
            compute(sl)

            @pl.when(k + NSUB < NCHUNK)
            def _():
                if swait_cond is None:
                    scatter_drain(other)
                else:
                    @pl.when(swait_cond)
                    def _():
                        scatter_drain(other)
                gather_start(other)

            pltpu.async_copy(mbuf, acc.at[dx], sems, add=True)

            @pl.when(k + 2 * NSUB < NCHUNK)
            def _():
                loads(i + 2, sl, dx2)

    loads(0, slot_a, didx0)
    loads(1, slot_b, didx1)
    gather_start(slot_a)

    def quad(j):
        process(4 * j, slot_a, slot_b, didx0, didx2, j >= 1)
        process(4 * j + 1, slot_b, slot_a, didx1, didx3, None)
        process(4 * j + 2, slot_a, slot_b, didx2, didx0, None)
        process(4 * j + 3, slot_b, slot_a, didx3, didx1, None)

    pl.loop(0, pl.cdiv(pl.cdiv(NCHUNK, NSUB), 4))(quad)

    scatter_drain(slot_a)
    scatter_drain(slot_b)
    plsc.subcore_barrier()
    pltpu.sync_copy(acc.at[pl.ds(zbase, nrows)],
                    a_hbm.at[c, pl.ds(zbase, nrows)])

    @pl.when(s == 0)
    def _():
        pltpu.sync_copy(acc.at[pl.ds(NSUB * nrows, ntail)],
                        a_hbm.at[c, pl.ds(NSUB * nrows, ntail)])


def _sc_layer(src, dst, hmv, r_arr, u16, zeros_acc):
    mesh = plsc.VectorSubcoreMesh(core_axis_name="c", subcore_axis_name="s")
    f = functools.partial(
        pl.kernel, _layer_body, mesh=mesh,
        out_type=jax.ShapeDtypeStruct((2, NN, 2 * CC), _F32),
        scratch_types=[pltpu.VMEM((6, CH), jnp.int32),
                       pltpu.VMEM((2, CH, CC), _F32),
                       pltpu.VMEM((2, CH * 16), _F32),
                       pltpu.VMEM((2, CH, 2 * CC), _F32),
                       pltpu.VMEM_SHARED((NN, 2 * CC), _F32),
                       pltpu.SemaphoreType.DMA,
                       pltpu.SemaphoreType.DMA,
                       pltpu.SemaphoreType.DMA,
                       pltpu.SemaphoreType.DMA,
                       pltpu.SemaphoreType.DMA,
                       pltpu.SemaphoreType.DMA,
                       pltpu.SemaphoreType.DMA,
                       pltpu.SemaphoreType.DMA],
    )
    return f()(src, dst, hmv, r_arr, u16.reshape(EE * 16), zeros_acc)


def _node0_body(spec_ref, we_ref, wm_ref, wv_ref, out_ref):
    oh = (spec_ref[...] == lax.broadcasted_iota(jnp.int32, (1, 16), 1))
    h0 = _dot(oh.astype(_F32), we_ref[...])
    out_ref[...] = jnp.concatenate(
        [_dot(h0, wm_ref[...]), _dot(h0, wv_ref[...])], axis=1)


def _tc_node0(spec, we16, wm, wv):
    return pl.pallas_call(
        _node0_body,
        grid=(NN // BN,),
        in_specs=[pl.BlockSpec((BN, 1), lambda i: (i, 0)),
                  pl.BlockSpec((16, CC), lambda i: (0, 0)),
                  pl.BlockSpec((CC, CC), lambda i: (0, 0)),
                  pl.BlockSpec((CC, CC), lambda i: (0, 0))],
        out_specs=pl.BlockSpec((BN, 2 * CC), lambda i: (i, 0)),
        out_shape=jax.ShapeDtypeStruct((NN, 2 * CC), _F32),
    )(spec, we16, wm, wv)


def _geom_body(vt_ref, wr1_ref, wr2_ref, r_ref, u_ref):
    v = vt_ref[...]
    r2 = jnp.sum(v * v, axis=0, keepdims=True) + 1e-12
    r = jnp.sqrt(r2)
    rinv = 1.0 / r
    uT = v * rinv

    def t8(a):
        return jnp.transpose(a, (1, 0))

    u_ref[...] = jnp.concatenate(
        [t8(uT), jnp.zeros((uT.shape[1], 8), _F32)], axis=1)
    x = r / RCUT
    nv = lax.broadcasted_iota(jnp.int32, (RBF, 1), 0).astype(_F32) + 1.0
    bes = jnp.sqrt(2.0 / RCUT) * jnp.sin(nv * (jnp.pi * x)) * rinv
    x6 = x * x * x
    x6 = x6 * x6
    fcut = (1.0 - 28.0 * x6 + 48.0 * x6 * x - 21.0 * x6 * x * x)
    fcut = jnp.where(x < 1.0, fcut, 0.0)
    rb = t8(bes * fcut)
    r_ref[...] = _dot(_silu(_dot(rb, wr1_ref[...])), wr2_ref[...])


def _tc_geom(vecT, wr1, wr2):
    return pl.pallas_call(
        _geom_body,
        grid=(EE // BE,),
        in_specs=[pl.BlockSpec((8, BE), lambda i: (0, i)),
                  pl.BlockSpec((RBF, CC), lambda i: (0, 0)),
                  pl.BlockSpec((CC, CC), lambda i: (0, 0))],
        out_specs=[pl.BlockSpec((BE, CC), lambda i: (i, 0)),
                   pl.BlockSpec((BE, 16), lambda i: (i, 0))],
        out_shape=[jax.ShapeDtypeStruct((EE, CC), _F32),
                   jax.ShapeDtypeStruct((EE, 16), _F32)],
    )(vecT, wr1, wr2)


def _node_body(a_ref, wu_ref, wm_ref, wv_ref, out_ref):
    a = a_ref[...] * (1.0 / AVGN)
    a1x = a[0, :, CC:]
    a1y = a[1, :, :CC]
    a1z = a[1, :, CC:]
    inv = a[0, :, :CC] + a1x * a1x + a1y * a1y + a1z * a1z
    h = _silu(_dot(inv, wu_ref[...]))
    out_ref[...] = jnp.concatenate(
        [_dot(h, wm_ref[...]), _dot(h, wv_ref[...])], axis=1)


def _tc_node(a_arr, wu, wm, wv):
    return pl.pallas_call(
        _node_body,
        grid=(NN // BN,),
        in_specs=[pl.BlockSpec((2, BN, 2 * CC), lambda i: (0, i, 0)),
                  pl.BlockSpec((CC, CC), lambda i: (0, 0)),
                  pl.BlockSpec((CC, CC), lambda i: (0, 0)),
                  pl.BlockSpec((CC, CC), lambda i: (0, 0))],
        out_specs=pl.BlockSpec((BN, 2 * CC), lambda i: (i, 0)),
        out_shape=jax.ShapeDtypeStruct((NN, 2 * CC), _F32),
    )(a_arr, wu, wm, wv)


def _final_body(a_ref, wu_ref, w1_ref, w2_ref, spec_ref, ae_ref, bat_ref,
                out_ref):
    @pl.when(pl.program_id(0) == 0)
    def _():
        out_ref[...] = jnp.zeros_like(out_ref)

    a = a_ref[...] * (1.0 / AVGN)
    a1x = a[0, :, CC:]
    a1y = a[1, :, :CC]
    a1z = a[1, :, CC:]
    inv = a[0, :, :CC] + a1x * a1x + a1y * a1y + a1z * a1z
    h = _silu(_dot(inv, wu_ref[...]))
    e = _dot(_silu(_dot(h, w1_ref[...])), w2_ref[...])
    oh_s = (spec_ref[...] == lax.broadcasted_iota(jnp.int32, (1, 16), 1))
    e = e + _dot(oh_s.astype(_F32), ae_ref[...])
    oh_b = (bat_ref[...] == lax.broadcasted_iota(jnp.int32, (1, 16), 1))
    out_ref[...] += jnp.sum(oh_b.astype(_F32) * e, axis=0, keepdims=True)


def _tc_final(a_arr, wu, w1, w2, spec, ae16, bat):
    return pl.pallas_call(
        _final_body,
        grid=(NN // BN,),
        in_specs=[pl.BlockSpec((2, BN, 2 * CC), lambda i: (0, i, 0)),
                  pl.BlockSpec((CC, CC), lambda i: (0, 0)),
                  pl.BlockSpec((CC, 16), lambda i: (0, 0)),
                  pl.BlockSpec((16, 1), lambda i: (0, 0)),
                  pl.BlockSpec((BN, 1), lambda i: (i, 0)),
                  pl.BlockSpec((16, 1), lambda i: (0, 0)),
                  pl.BlockSpec((BN, 1), lambda i: (i, 0))],
        out_specs=pl.BlockSpec((1, 16), lambda i: (0, 0)),
        out_shape=jax.ShapeDtypeStruct((1, 16), _F32),
    )(a_arr, wu, w1, w2, spec, ae16, bat)


def kernel(positions, edge_index, species, batch,
           W_embed, W_r1, W_r2,
           W_msg0, W_vec0, W_up0,
           W_msg1, W_vec1, W_up1,
           W_ro1, W_ro2, atomic_E):
    src = edge_index[0].astype(jnp.int32)
    dst = edge_index[1].astype(jnp.int32)
    pos1d = jnp.pad(positions.astype(_F32), ((0, 0), (0, 5))).reshape(-1)
    we16 = jnp.pad(W_embed, ((0, 6), (0, 0)))
    ae16 = jnp.pad(atomic_E, (0, 6)).reshape(16, 1)
    spec = species.astype(jnp.int32).reshape(NN, 1)
    bat = batch.astype(jnp.int32).reshape(NN, 1)
    zeros_acc = jnp.zeros((NN, 2 * CC), _F32)

    hmv = _tc_node0(spec, we16, W_msg0, W_vec0)
    vecT = _sc_vec(src, dst, pos1d)
    r_arr, u8 = _tc_geom(vecT, W_r1, W_r2)

    a_arr = _sc_layer(src, dst, hmv, r_arr, u8, zeros_acc)
    hmv = _tc_node(a_arr, W_up0, W_msg1, W_vec1)
    a_arr = _sc_layer(src, dst, hmv, r_arr, u8, zeros_acc)

    energy = _tc_final(a_arr, W_up1, W_ro1, W_ro2, spec, ae16, bat)
    return energy.reshape(NGR)


# speedup vs baseline: 2.0114x; 1.0402x over previous
import functools

import jax
import jax.numpy as jnp
from jax import lax
from jax.experimental import pallas as pl
from jax.experimental.pallas import tpu as pltpu
from jax.experimental.pallas import tpu_sc as plsc

NN = 10000
EE = 160000
CC = 64
RBF = 8
NGR = 16
RCUT = 5.0
AVGN = 16.0

NCORE = 2
NSUB = 16

CH = 64
NCHUNK = EE // CH

BE = 3200
BN = 1000

_F32 = jnp.float32


def _dot(a, b):
    return lax.dot_general(a, b, (((1,), (0,)), ((), ())),
                           preferred_element_type=_F32,
                           precision=lax.Precision.DEFAULT)


def _silu(x):
    return x * (1.0 / (1.0 + jnp.exp(-x)))


VCH = 1280
NVCH = EE // VCH


def _vec_body(src_hbm, dst_hbm, pos_hbm, vt_hbm, pos_v, sidx, didx, vbuf):
    c = lax.axis_index("c")
    s = lax.axis_index("s")
    wid = s * NCORE + c

    pltpu.sync_copy(pos_hbm, pos_v)
    zero16 = jnp.zeros((16,), _F32)

    def zrow(g, _=None):
        for j in range(3, 8):
            vbuf[j, pl.ds(g * 16, 16)] = zero16

    plsc.parallel_loop(0, VCH // 16, unroll=4)(zrow)

    def chunk(i):
        k = i * (NCORE * NSUB) + wid

        @pl.when(k < NVCH)
        def _():
            base = k * VCH
            pltpu.sync_copy(src_hbm.at[pl.ds(base, VCH)], sidx)
            pltpu.sync_copy(dst_hbm.at[pl.ds(base, VCH)], didx)

            def grp(g, _=None):
                sv = sidx[pl.ds(g * 16, 16)] * 8
                dv = didx[pl.ds(g * 16, 16)] * 8
                for j in range(3):
                    ps = plsc.load_gather(pos_v, [sv + j])
                    pd = plsc.load_gather(pos_v, [dv + j])
                    vbuf[j, pl.ds(g * 16, 16)] = pd - ps

            plsc.parallel_loop(0, VCH // 16, unroll=4)(grp)
            pltpu.sync_copy(vbuf, vt_hbm.at[:, pl.ds(base, VCH)])

    pl.loop(0, pl.cdiv(NVCH, NCORE * NSUB))(chunk)


def _sc_vec(src, dst, pos1d):
    mesh = plsc.VectorSubcoreMesh(core_axis_name="c", subcore_axis_name="s")
    f = functools.partial(
        pl.kernel, _vec_body, mesh=mesh,
        out_type=jax.ShapeDtypeStruct((8, EE), _F32),
        scratch_types=[pltpu.VMEM((NN * 8,), _F32),
                       pltpu.VMEM((VCH,), jnp.int32),
                       pltpu.VMEM((VCH,), jnp.int32),
                       pltpu.VMEM((8, VCH), _F32)],
        compiler_params=pltpu.CompilerParams(needs_layout_passes=False),
    )
    return f()(src, dst, pos1d)


def _layer_body(src_hbm, dst_hbm, hmv_hbm, r_hbm, u_hbm, zeros_hbm, a_hbm,
                idx6, rbuf2, ubuf2, hsb2,
                acc,
                semi_a, semi_b, seml_a, seml_b,
                semg_a, semg_b, sems_a, sems_b):
    c = lax.axis_index("c")
    s = lax.axis_index("s")
    sidx_a, sidx_b = idx6.at[0], idx6.at[1]
    didx0, didx1, didx2, didx3 = (idx6.at[2], idx6.at[3],
                                  idx6.at[4], idx6.at[5])
    rbuf_a, rbuf_b = rbuf2.at[0], rbuf2.at[1]
    ubuf_a, ubuf_b = ubuf2.at[0], ubuf2.at[1]
    hsb_a, hsb_b = hsb2.at[0], hsb2.at[1]
    mbuf_a, mbuf_b = hsb_a, hsb_b
    nrows = 624
    ntail = NN - NSUB * nrows

    zbase = s * nrows
    pltpu.sync_copy(zeros_hbm.at[pl.ds(zbase, nrows)],
                    acc.at[pl.ds(zbase, nrows)])

    @pl.when(s == 0)
    def _():
        pltpu.sync_copy(zeros_hbm.at[pl.ds(NSUB * nrows, ntail)],
                        acc.at[pl.ds(NSUB * nrows, ntail)])

    plsc.subcore_barrier()

    slot_a = (sidx_a, rbuf_a, ubuf_a, hsb_a, mbuf_a,
              semi_a, seml_a, semg_a, sems_a)
    slot_b = (sidx_b, rbuf_b, ubuf_b, hsb_b, mbuf_b,
              semi_b, seml_b, semg_b, sems_b)

    def loads(i, sl, dx):
        sidx, rbuf, ubuf, _, _, semi, seml, _, _ = sl
        k = i * NSUB + s
        base = k * CH
        pltpu.async_copy(src_hbm.at[pl.ds(base, CH)], sidx, semi)
        pltpu.async_copy(dst_hbm.at[pl.ds(base, CH)], dx, seml)
        pltpu.async_copy(r_hbm.at[pl.ds(base, CH)], rbuf, seml)
        pltpu.async_copy(u_hbm.at[pl.ds(base, CH)], ubuf, seml)

    def gather_start(sl):
        sidx, _, _, hsb, _, semi, _, semg, _ = sl
        pltpu.make_async_copy(src_hbm.at[pl.ds(0, CH)], sidx, semi).wait()
        pltpu.async_copy(hmv_hbm.at[sidx], hsb, semg)

    def scatter_drain(sl):
        mbuf, sems = sl[4], sl[8]
        pltpu.make_async_copy(mbuf, acc.at[idx6.at[2]], sems).wait()

    def compute(sl):
        _, rbuf, ubuf, hsb, mbuf, _, _, _, _ = sl

        @pl.when(c == 0)
        def _():
            def edge(e, _=None):
                uvec = ubuf[e, pl.ds(0, 16)]
                ux = uvec[0]
                for q in range(CC // 16):
                    rv = rbuf[e, pl.ds(q * 16, 16)]
                    hm = hsb[e, pl.ds(q * 16, 16)]
                    hv = hsb[e, pl.ds(CC + q * 16, 16)]
                    mbuf[e, pl.ds(q * 16, 16)] = rv * hm
                    mbuf[e, pl.ds(CC + q * 16, 16)] = rv * hv * ux

            plsc.parallel_loop(0, CH, unroll=4)(edge)

        @pl.when(c == 1)
        def _():
            def edge(e, _=None):
                uvec = ubuf[e, pl.ds(0, 16)]
                uy = uvec[1]
                uz = uvec[2]
                for q in range(CC // 16):
                    rv = rbuf[e, pl.ds(q * 16, 16)]
                    hv = hsb[e, pl.ds(CC + q * 16, 16)]
                    t = rv * hv
                    mbuf[e, pl.ds(q * 16, 16)] = t * uy
                    mbuf[e, pl.ds(CC + q * 16, 16)] = t * uz

            plsc.parallel_loop(0, CH, unroll=4)(edge)

    def process(i, sl, other, dx, dx2, swait_cond):
        _, rbuf, ubuf, hsb, mbuf, _, seml, semg, sems = sl
        k = i * NSUB + s

        @pl.when(k < NCHUNK)
        def _():
            pltpu.make_async_copy(dst_hbm.at[pl.ds(0, CH)], dx, seml).wait()
            pltpu.make_async_copy(r_hbm.at[pl.ds(0, CH)], rbuf, seml).wait()
            pltpu.make_async_copy(u_hbm.at[pl.ds(0, CH)], ubuf, seml).wait()
            pltpu.make_async_copy(hmv_hbm.at[sl[0]], hsb, semg).wait()
            compute(sl)

            @pl.when(k + NSUB < NCHUNK)
            def _():
                if swait_cond is None:
                    scatter_drain(other)
                else:
                    @pl.when(swait_cond)
                    def _():
                        scatter_drain(other)
                gather_start(other)

            pltpu.async_copy(mbuf, acc.at[dx], sems, add=True)

            @pl.when(k + 2 * NSUB < NCHUNK)
            def _():
                loads(i + 2, sl, dx2)

    loads(0, slot_a, didx0)
    loads(1, slot_b, didx1)
    gather_start(slot_a)

    def quad(j):
        process(4 * j, slot_a, slot_b, didx0, didx2, j >= 1)
        process(4 * j + 1, slot_b, slot_a, didx1, didx3, None)
        process(4 * j + 2, slot_a, slot_b, didx2, didx0, None)
        process(4 * j + 3, slot_b, slot_a, didx3, didx1, None)

    pl.loop(0, pl.cdiv(pl.cdiv(NCHUNK, NSUB), 4))(quad)

    scatter_drain(slot_a)
    scatter_drain(slot_b)
    plsc.subcore_barrier()
    pltpu.sync_copy(acc.at[pl.ds(zbase, nrows)],
                    a_hbm.at[c, pl.ds(zbase, nrows)])

    @pl.when(s == 0)
    def _():
        pltpu.sync_copy(acc.at[pl.ds(NSUB * nrows, ntail)],
                        a_hbm.at[c, pl.ds(NSUB * nrows, ntail)])


def _sc_layer(src, dst, hmv, r_arr, u16, zeros_acc):
    mesh = plsc.VectorSubcoreMesh(core_axis_name="c", subcore_axis_name="s")
    f = functools.partial(
        pl.kernel, _layer_body, mesh=mesh,
        out_type=jax.ShapeDtypeStruct((2, NN, 2 * CC), _F32),
        scratch_types=[pltpu.VMEM((6, CH), jnp.int32),
                       pltpu.VMEM((2, CH, CC), _F32),
                       pltpu.VMEM((2, CH, 16), _F32),
                       pltpu.VMEM((2, CH, 2 * CC), _F32),
                       pltpu.VMEM_SHARED((NN, 2 * CC), _F32),
                       pltpu.SemaphoreType.DMA,
                       pltpu.SemaphoreType.DMA,
                       pltpu.SemaphoreType.DMA,
                       pltpu.SemaphoreType.DMA,
                       pltpu.SemaphoreType.DMA,
                       pltpu.SemaphoreType.DMA,
                       pltpu.SemaphoreType.DMA,
                       pltpu.SemaphoreType.DMA],
    )
    return f()(src, dst, hmv, r_arr, u16, zeros_acc)


def _node0_body(spec_ref, we_ref, wm_ref, wv_ref, out_ref):
    oh = (spec_ref[...] == lax.broadcasted_iota(jnp.int32, (1, 16), 1))
    h0 = _dot(oh.astype(_F32), we_ref[...])
    out_ref[...] = jnp.concatenate(
        [_dot(h0, wm_ref[...]), _dot(h0, wv_ref[...])], axis=1)


def _tc_node0(spec, we16, wm, wv):
    return pl.pallas_call(
        _node0_body,
        grid=(NN // BN,),
        in_specs=[pl.BlockSpec((BN, 1), lambda i: (i, 0)),
                  pl.BlockSpec((16, CC), lambda i: (0, 0)),
                  pl.BlockSpec((CC, CC), lambda i: (0, 0)),
                  pl.BlockSpec((CC, CC), lambda i: (0, 0))],
        out_specs=pl.BlockSpec((BN, 2 * CC), lambda i: (i, 0)),
        out_shape=jax.ShapeDtypeStruct((NN, 2 * CC), _F32),
    )(spec, we16, wm, wv)


def _geom_body(vt_ref, wr1_ref, wr2_ref, r_ref, u_ref):
    v = vt_ref[...]
    r2 = jnp.sum(v * v, axis=0, keepdims=True) + 1e-12
    r = jnp.sqrt(r2)
    rinv = 1.0 / r
    uT = v * rinv

    def t8(a):
        return jnp.transpose(a, (1, 0))

    u_ref[...] = jnp.concatenate(
        [t8(uT), jnp.zeros((uT.shape[1], 8), _F32)], axis=1)
    x = r / RCUT
    nv = lax.broadcasted_iota(jnp.int32, (RBF, 1), 0).astype(_F32) + 1.0
    bes = jnp.sqrt(2.0 / RCUT) * jnp.sin(nv * (jnp.pi * x)) * rinv
    x6 = x * x * x
    x6 = x6 * x6
    fcut = (1.0 - 28.0 * x6 + 48.0 * x6 * x - 21.0 * x6 * x * x)
    fcut = jnp.where(x < 1.0, fcut, 0.0)
    rb = t8(bes * fcut)
    r_ref[...] = _dot(_silu(_dot(rb, wr1_ref[...])), wr2_ref[...])


def _tc_geom(vecT, wr1, wr2):
    return pl.pallas_call(
        _geom_body,
        grid=(EE // BE,),
        in_specs=[pl.BlockSpec((8, BE), lambda i: (0, i)),
                  pl.BlockSpec((RBF, CC), lambda i: (0, 0)),
                  pl.BlockSpec((CC, CC), lambda i: (0, 0))],
        out_specs=[pl.BlockSpec((BE, CC), lambda i: (i, 0)),
                   pl.BlockSpec((BE, 16), lambda i: (i, 0))],
        out_shape=[jax.ShapeDtypeStruct((EE, CC), _F32),
                   jax.ShapeDtypeStruct((EE, 16), _F32)],
    )(vecT, wr1, wr2)


def _node_body(a_ref, wu_ref, wm_ref, wv_ref, out_ref):
    a = a_ref[...] * (1.0 / AVGN)
    a1x = a[0, :, CC:]
    a1y = a[1, :, :CC]
    a1z = a[1, :, CC:]
    inv = a[0, :, :CC] + a1x * a1x + a1y * a1y + a1z * a1z
    h = _silu(_dot(inv, wu_ref[...]))
    out_ref[...] = jnp.concatenate(
        [_dot(h, wm_ref[...]), _dot(h, wv_ref[...])], axis=1)


def _tc_node(a_arr, wu, wm, wv):
    return pl.pallas_call(
        _node_body,
        grid=(NN // BN,),
        in_specs=[pl.BlockSpec((2, BN, 2 * CC), lambda i: (0, i, 0)),
                  pl.BlockSpec((CC, CC), lambda i: (0, 0)),
                  pl.BlockSpec((CC, CC), lambda i: (0, 0)),
                  pl.BlockSpec((CC, CC), lambda i: (0, 0))],
        out_specs=pl.BlockSpec((BN, 2 * CC), lambda i: (i, 0)),
        out_shape=jax.ShapeDtypeStruct((NN, 2 * CC), _F32),
    )(a_arr, wu, wm, wv)


def _final_body(a_ref, wu_ref, w1_ref, w2_ref, spec_ref, ae_ref, bat_ref,
                out_ref):
    @pl.when(pl.program_id(0) == 0)
    def _():
        out_ref[...] = jnp.zeros_like(out_ref)

    a = a_ref[...] * (1.0 / AVGN)
    a1x = a[0, :, CC:]
    a1y = a[1, :, :CC]
    a1z = a[1, :, CC:]
    inv = a[0, :, :CC] + a1x * a1x + a1y * a1y + a1z * a1z
    h = _silu(_dot(inv, wu_ref[...]))
    e = _dot(_silu(_dot(h, w1_ref[...])), w2_ref[...])
    oh_s = (spec_ref[...] == lax.broadcasted_iota(jnp.int32, (1, 16), 1))
    e = e + _dot(oh_s.astype(_F32), ae_ref[...])
    oh_b = (bat_ref[...] == lax.broadcasted_iota(jnp.int32, (1, 16), 1))
    out_ref[...] += jnp.sum(oh_b.astype(_F32) * e, axis=0, keepdims=True)


def _tc_final(a_arr, wu, w1, w2, spec, ae16, bat):
    return pl.pallas_call(
        _final_body,
        grid=(NN // BN,),
        in_specs=[pl.BlockSpec((2, BN, 2 * CC), lambda i: (0, i, 0)),
                  pl.BlockSpec((CC, CC), lambda i: (0, 0)),
                  pl.BlockSpec((CC, 16), lambda i: (0, 0)),
                  pl.BlockSpec((16, 1), lambda i: (0, 0)),
                  pl.BlockSpec((BN, 1), lambda i: (i, 0)),
                  pl.BlockSpec((16, 1), lambda i: (0, 0)),
                  pl.BlockSpec((BN, 1), lambda i: (i, 0))],
        out_specs=pl.BlockSpec((1, 16), lambda i: (0, 0)),
        out_shape=jax.ShapeDtypeStruct((1, 16), _F32),
    )(a_arr, wu, w1, w2, spec, ae16, bat)


def kernel(positions, edge_index, species, batch,
           W_embed, W_r1, W_r2,
           W_msg0, W_vec0, W_up0,
           W_msg1, W_vec1, W_up1,
           W_ro1, W_ro2, atomic_E):
    src = edge_index[0].astype(jnp.int32)
    dst = edge_index[1].astype(jnp.int32)
    pos1d = jnp.pad(positions.astype(_F32), ((0, 0), (0, 5))).reshape(-1)
    we16 = jnp.pad(W_embed, ((0, 6), (0, 0)))
    ae16 = jnp.pad(atomic_E, (0, 6)).reshape(16, 1)
    spec = species.astype(jnp.int32).reshape(NN, 1)
    bat = batch.astype(jnp.int32).reshape(NN, 1)
    zeros_acc = jnp.zeros((NN, 2 * CC), _F32)

    hmv = _tc_node0(spec, we16, W_msg0, W_vec0)
    vecT = _sc_vec(src, dst, pos1d)
    r_arr, u8 = _tc_geom(vecT, W_r1, W_r2)

    a_arr = _sc_layer(src, dst, hmv, r_arr, u8, zeros_acc)
    hmv = _tc_node(a_arr, W_up0, W_msg1, W_vec1)
    a_arr = _sc_layer(src, dst, hmv, r_arr, u8, zeros_acc)

    energy = _tc_final(a_arr, W_up1, W_ro1, W_ro2, spec, ae16, bat)
    return energy.reshape(NGR)
